# Initial kernel scaffold; baseline (speedup 1.0000x reference)
import jax
import jax.numpy as jnp
from jax.experimental import pallas as pl


def kernel(x, tables):
    raise NotImplementedError("write your pallas kernel here")


# speedup vs baseline: 1.3215x; 1.3215x over previous
import functools
import math

import jax
import jax.numpy as jnp
from jax import lax
from jax.experimental import pallas as pl
from jax.experimental.pallas import tpu as pltpu
from jax.experimental.pallas import tpu_sc as plsc

_NUM_LEVELS = 16
_FEAT = 2
_LOG2_HASH = 19
_HASH_SIZE = 1 << _LOG2_HASH
_MASK = _HASH_SIZE - 1
_BASE_RES = 16
_FINEST_RES = 512
_B = math.exp((math.log(_FINEST_RES) - math.log(_BASE_RES)) / (_NUM_LEVELS - 1))
_RES = [max(int(math.floor(_BASE_RES * (_B ** l))), 1) for l in range(_NUM_LEVELS)]
_C0, _C1, _C2 = 1540863946, 1257487969, 1034312349

_N = 524288
_NW = 32
_P = 1024
_PER_W = _N // _NW
_CHUNKS = _PER_W // _P
_G = _P // 16
_K = 8 * _P


def _floor_f32(v):
    ti = v.astype(jnp.int32)
    tf = ti.astype(jnp.float32)
    bi = jnp.where(tf > v, ti - 1, ti)
    return bi, v - bi.astype(jnp.float32)


def _body(xs_hbm, ys_hbm, zs_hbm, tab_hbm, out_hbm,
          xs, ys, zs, idx0_v, idx1_v, w_v, rows0_v, rows1_v, outbuf, sem):
    nc = 2
    wid = lax.axis_index("s") * nc + lax.axis_index("c")
    iota = lax.iota(jnp.int32, 16)
    fzero = jnp.full((16,), 0, jnp.int32)
    fone = jnp.full((16,), 1, jnp.int32)

    def chunk_body(ch, _):
        base = wid * jnp.int32(_PER_W) + ch * jnp.int32(_P)
        pltpu.sync_copy(xs_hbm.at[pl.ds(base, _P)], xs)
        pltpu.sync_copy(ys_hbm.at[pl.ds(base, _P)], ys)
        pltpu.sync_copy(zs_hbm.at[pl.ds(base, _P)], zs)

        for lvl in range(_NUM_LEVELS):
            res = _RES[lvl]
            lvl_off = lvl * _HASH_SIZE

            def pass1(g, _, res=res, lvl_off=lvl_off):
                off = g * jnp.int32(16)
                xv = xs[pl.ds(off, 16)]
                yv = ys[pl.ds(off, 16)]
                zv = zs[pl.ds(off, 16)]
                bx, tx = _floor_f32(xv * res - 0.5)
                by, ty = _floor_f32(yv * res - 0.5)
                bz, tz = _floor_f32(zv * res - 0.5)
                c0, c1, c2 = jnp.int32(_C0), jnp.int32(_C1), jnp.int32(_C2)
                px = (bx * c0, bx * c0 + c0)
                py = (by * c1, by * c1 + c1)
                pz = (bz * c2, bz * c2 + c2)
                wx = (1.0 - tx, tx)
                wy = (1.0 - ty, ty)
                wz = (1.0 - tz, tz)
                c = 0
                for ox in range(2):
                    for oy in range(2):
                        pxy = px[ox] ^ py[oy]
                        wxy = wx[ox] * wy[oy]
                        for oz in range(2):
                            h = ((pxy ^ pz[oz]) & jnp.int32(_MASK)) + jnp.int32(lvl_off)
                            h2 = h * jnp.int32(2)
                            idx0_v[pl.ds(jnp.int32(c * _P) + off, 16)] = h2
                            idx1_v[pl.ds(jnp.int32(c * _P) + off, 16)] = h2 + jnp.int32(1)
                            w_v[pl.ds(jnp.int32(c * _P) + off, 16)] = wxy * wz[oz]
                            c += 1
                return jnp.int32(0)

            lax.fori_loop(jnp.int32(0), jnp.int32(_G), pass1, jnp.int32(0),
                          unroll=False)

            cp0 = pltpu.async_copy(tab_hbm.at[idx0_v], rows0_v, sem)
            cp1 = pltpu.async_copy(tab_hbm.at[idx1_v], rows1_v, sem)
            cp0.wait()
            cp1.wait()

            def pass2(g, _, lvl=lvl):
                off = g * jnp.int32(16)
                acc0 = jnp.full((16,), 0.0, jnp.float32)
                acc1 = jnp.full((16,), 0.0, jnp.float32)
                for c in range(8):
                    w16 = w_v[pl.ds(jnp.int32(c * _P) + off, 16)]
                    r0 = rows0_v[pl.ds(jnp.int32(c * _P) + off, 16)]
                    r1 = rows1_v[pl.ds(jnp.int32(c * _P) + off, 16)]
                    acc0 = acc0 + w16 * r0
                    acc1 = acc1 + w16 * r1
                opos = (off + iota) * jnp.int32(2 * _NUM_LEVELS) + jnp.int32(2 * lvl)
                plsc.store_scatter(outbuf, [opos], acc0)
                plsc.store_scatter(outbuf, [opos + jnp.int32(1)], acc1)
                return jnp.int32(0)

            lax.fori_loop(jnp.int32(0), jnp.int32(_G), pass2, jnp.int32(0),
                          unroll=False)

        pltpu.sync_copy(outbuf, out_hbm.at[pl.ds(base * jnp.int32(2 * _NUM_LEVELS),
                                                 _P * 2 * _NUM_LEVELS)])
        return jnp.int32(0)

    lax.fori_loop(jnp.int32(0), jnp.int32(_CHUNKS), chunk_body, jnp.int32(0),
                  unroll=False)


@jax.jit
def kernel(x, tables):
    xt = x.T
    tab = tables.reshape(_NUM_LEVELS * _HASH_SIZE * _FEAT)
    run = pl.kernel(
        _body,
        out_type=jax.ShapeDtypeStruct((_N * 2 * _NUM_LEVELS,), jnp.float32),
        mesh=plsc.VectorSubcoreMesh(core_axis_name="c", subcore_axis_name="s"),
        compiler_params=pltpu.CompilerParams(needs_layout_passes=False),
        scratch_types=[
            pltpu.VMEM((_P,), jnp.float32),
            pltpu.VMEM((_P,), jnp.float32),
            pltpu.VMEM((_P,), jnp.float32),
            pltpu.VMEM((_K,), jnp.int32),
            pltpu.VMEM((_K,), jnp.int32),
            pltpu.VMEM((_K,), jnp.float32),
            pltpu.VMEM((_K,), jnp.float32),
            pltpu.VMEM((_K,), jnp.float32),
            pltpu.VMEM((_P * 2 * _NUM_LEVELS,), jnp.float32),
            pltpu.SemaphoreType.DMA,
        ],
    )
    out = run(xt[0], xt[1], xt[2], tab)
    return out.reshape(_N, _NUM_LEVELS, _FEAT)


# speedup vs baseline: 1.3821x; 1.0458x over previous
import math

import jax
import jax.numpy as jnp
from jax import lax
from jax.experimental import pallas as pl
from jax.experimental.pallas import tpu as pltpu
from jax.experimental.pallas import tpu_sc as plsc

_NUM_LEVELS = 16
_FEAT = 2
_LOG2_HASH = 19
_HASH_SIZE = 1 << _LOG2_HASH
_MASK = _HASH_SIZE - 1
_BASE_RES = 16
_FINEST_RES = 512
_B = math.exp((math.log(_FINEST_RES) - math.log(_BASE_RES)) / (_NUM_LEVELS - 1))
_RES = [max(int(math.floor(_BASE_RES * (_B ** l))), 1) for l in range(_NUM_LEVELS)]
_C0, _C1, _C2 = 1540863946, 1257487969, 1034312349

_N = 524288
_NW = 32
_P = 1024
_PER_W = _N // _NW
_CHUNKS = _PER_W // _P
_G = _P // 16
_K = 8 * _P


def _floor_f32(v):
    ti = v.astype(jnp.int32)
    tf = ti.astype(jnp.float32)
    bi = jnp.where(tf > v, ti - 1, ti)
    return bi, v - bi.astype(jnp.float32)


def _make_pass1(xs, ys, zs, idx0_v, idx1_v, w_v, res, lvl_off):
    def pass1(g, _):
        off = g * jnp.int32(16)
        xv = xs[pl.ds(off, 16)]
        yv = ys[pl.ds(off, 16)]
        zv = zs[pl.ds(off, 16)]
        bx, tx = _floor_f32(xv * res - 0.5)
        by, ty = _floor_f32(yv * res - 0.5)
        bz, tz = _floor_f32(zv * res - 0.5)
        c0, c1, c2 = jnp.int32(_C0), jnp.int32(_C1), jnp.int32(_C2)
        px = (bx * c0, bx * c0 + c0)
        py = (by * c1, by * c1 + c1)
        pz = (bz * c2, bz * c2 + c2)
        wx = (1.0 - tx, tx)
        wy = (1.0 - ty, ty)
        wz = (1.0 - tz, tz)
        c = 0
        for ox in range(2):
            for oy in range(2):
                pxy = px[ox] ^ py[oy]
                wxy = wx[ox] * wy[oy]
                for oz in range(2):
                    h = ((pxy ^ pz[oz]) & jnp.int32(_MASK)) + jnp.int32(lvl_off)
                    h2 = h * jnp.int32(2)
                    idx0_v[pl.ds(jnp.int32(c * _P) + off, 16)] = h2
                    idx1_v[pl.ds(jnp.int32(c * _P) + off, 16)] = h2 + jnp.int32(1)
                    w_v[pl.ds(jnp.int32(c * _P) + off, 16)] = wxy * wz[oz]
                    c += 1
        return jnp.int32(0)
    return pass1


def _make_pass2(w_v, rows0_v, rows1_v, outbuf, iota, lvl):
    def pass2(g, _):
        off = g * jnp.int32(16)
        acc0 = jnp.full((16,), 0.0, jnp.float32)
        acc1 = jnp.full((16,), 0.0, jnp.float32)
        for c in range(8):
            w16 = w_v[pl.ds(jnp.int32(c * _P) + off, 16)]
            r0 = rows0_v[pl.ds(jnp.int32(c * _P) + off, 16)]
            r1 = rows1_v[pl.ds(jnp.int32(c * _P) + off, 16)]
            acc0 = acc0 + w16 * r0
            acc1 = acc1 + w16 * r1
        opos = (off + iota) * jnp.int32(2 * _NUM_LEVELS) + jnp.int32(2 * lvl)
        plsc.store_scatter(outbuf, [opos], acc0)
        plsc.store_scatter(outbuf, [opos + jnp.int32(1)], acc1)
        return jnp.int32(0)
    return pass2


def _body(xs_hbm, ys_hbm, zs_hbm, tab_hbm, out_hbm,
          xs, ys, zs,
          idx0_a, idx1_a, w_a, rows0_a, rows1_a,
          idx0_b, idx1_b, w_b, rows0_b, rows1_b,
          outbuf, sem_a, sem_b):
    nc = 2
    wid = lax.axis_index("s") * nc + lax.axis_index("c")
    iota = lax.iota(jnp.int32, 16)
    sets = ((idx0_a, idx1_a, w_a, rows0_a, rows1_a, sem_a),
            (idx0_b, idx1_b, w_b, rows0_b, rows1_b, sem_b))

    def chunk_body(ch, _):
        base = wid * jnp.int32(_PER_W) + ch * jnp.int32(_P)
        pltpu.sync_copy(xs_hbm.at[pl.ds(base, _P)], xs)
        pltpu.sync_copy(ys_hbm.at[pl.ds(base, _P)], ys)
        pltpu.sync_copy(zs_hbm.at[pl.ds(base, _P)], zs)

        pend = None
        for lvl in range(_NUM_LEVELS):
            idx0_v, idx1_v, w_v, rows0_v, rows1_v, sem = sets[lvl % 2]
            pass1 = _make_pass1(xs, ys, zs, idx0_v, idx1_v, w_v,
                                _RES[lvl], lvl * _HASH_SIZE)
            lax.fori_loop(jnp.int32(0), jnp.int32(_G), pass1, jnp.int32(0),
                          unroll=False)
            cp0 = pltpu.async_copy(tab_hbm.at[idx0_v], rows0_v, sem)
            cp1 = pltpu.async_copy(tab_hbm.at[idx1_v], rows1_v, sem)
            if pend is not None:
                plvl, pcp0, pcp1, pw, prows0, prows1 = pend
                pcp0.wait()
                pcp1.wait()
                pass2 = _make_pass2(pw, prows0, prows1, outbuf, iota, plvl)
                lax.fori_loop(jnp.int32(0), jnp.int32(_G), pass2,
                              jnp.int32(0), unroll=False)
            pend = (lvl, cp0, cp1, w_v, rows0_v, rows1_v)

        plvl, pcp0, pcp1, pw, prows0, prows1 = pend
        pcp0.wait()
        pcp1.wait()
        pass2 = _make_pass2(pw, prows0, prows1, outbuf, iota, plvl)
        lax.fori_loop(jnp.int32(0), jnp.int32(_G), pass2, jnp.int32(0),
                      unroll=False)

        pltpu.sync_copy(outbuf, out_hbm.at[pl.ds(base * jnp.int32(2 * _NUM_LEVELS),
                                                 _P * 2 * _NUM_LEVELS)])
        return jnp.int32(0)

    lax.fori_loop(jnp.int32(0), jnp.int32(_CHUNKS), chunk_body, jnp.int32(0),
                  unroll=False)


@jax.jit
def kernel(x, tables):
    xt = x.T
    tab = tables.reshape(_NUM_LEVELS * _HASH_SIZE * _FEAT)
    buf = lambda dt: pltpu.VMEM((_K,), dt)
    run = pl.kernel(
        _body,
        out_type=jax.ShapeDtypeStruct((_N * 2 * _NUM_LEVELS,), jnp.float32),
        mesh=plsc.VectorSubcoreMesh(core_axis_name="c", subcore_axis_name="s"),
        compiler_params=pltpu.CompilerParams(needs_layout_passes=False),
        scratch_types=[
            pltpu.VMEM((_P,), jnp.float32),
            pltpu.VMEM((_P,), jnp.float32),
            pltpu.VMEM((_P,), jnp.float32),
            buf(jnp.int32), buf(jnp.int32), buf(jnp.float32),
            buf(jnp.float32), buf(jnp.float32),
            buf(jnp.int32), buf(jnp.int32), buf(jnp.float32),
            buf(jnp.float32), buf(jnp.float32),
            pltpu.VMEM((_P * 2 * _NUM_LEVELS,), jnp.float32),
            pltpu.SemaphoreType.DMA,
            pltpu.SemaphoreType.DMA,
        ],
    )
    out = run(xt[0], xt[1], xt[2], tab)
    return out.reshape(_N, _NUM_LEVELS, _FEAT)


# speedup vs baseline: 7.2474x; 5.2439x over previous
import math

import jax
import jax.numpy as jnp
from jax import lax
from jax.experimental import pallas as pl
from jax.experimental.pallas import tpu as pltpu
from jax.experimental.pallas import tpu_sc as plsc

_NUM_LEVELS = 16
_FEAT = 2
_LOG2_HASH = 19
_HASH_SIZE = 1 << _LOG2_HASH
_MASK = _HASH_SIZE - 1
_BASE_RES = 16
_FINEST_RES = 512
_B = math.exp((math.log(_FINEST_RES) - math.log(_BASE_RES)) / (_NUM_LEVELS - 1))
_RES = [max(int(math.floor(_BASE_RES * (_B ** l))), 1) for l in range(_NUM_LEVELS)]
_C0, _C1, _C2 = 1540863946, 1257487969, 1034312349

_N = 524288
_NW = 32
_P = 512
_PER_W = _N // _NW
_CHUNKS = _PER_W // _P
_G = _P // 16
_K = 8 * _P

_NCACHE = 3
_GRID_R = [_RES[l] + 2 for l in range(_NCACHE)]
_GRID_SZ = [r * r * r for r in _GRID_R]
_GRID_PAD = [(sz + 7) // 8 * 8 for sz in _GRID_SZ]
_GRID_M = [(1 << 20) // r + 1 for r in _GRID_R]


def _floor_f32(v):
    ti = v.astype(jnp.int32)
    tf = ti.astype(jnp.float32)
    bi = jnp.where(tf > v, ti - 1, ti)
    return bi, v - bi.astype(jnp.float32)


def _tiled_entry(h, lvl):
    t = h & jnp.int32(127)
    return ((h - t) << 1) + t + jnp.int32(lvl * 2 * _HASH_SIZE)


def _hash3(bx, by, bz):
    c0, c1, c2 = jnp.int32(_C0), jnp.int32(_C1), jnp.int32(_C2)
    return ((bx * c0) ^ (by * c1) ^ (bz * c2)) & jnp.int32(_MASK)


def _make_pass1(xs, ys, zs, idx0_v, idx1_v, res, lvl):
    def pass1(g, _):
        off = g * jnp.int32(16)
        bx, _tx = _floor_f32(xs[pl.ds(off, 16)] * res - 0.5)
        by, _ty = _floor_f32(ys[pl.ds(off, 16)] * res - 0.5)
        bz, _tz = _floor_f32(zs[pl.ds(off, 16)] * res - 0.5)
        c0, c1, c2 = jnp.int32(_C0), jnp.int32(_C1), jnp.int32(_C2)
        px = (bx * c0, bx * c0 + c0)
        py = (by * c1, by * c1 + c1)
        pz = (bz * c2, bz * c2 + c2)
        c = 0
        for ox in range(2):
            for oy in range(2):
                pxy = px[ox] ^ py[oy]
                for oz in range(2):
                    h = (pxy ^ pz[oz]) & jnp.int32(_MASK)
                    e = _tiled_entry(h, lvl)
                    idx0_v[pl.ds(jnp.int32(c * _P) + off, 16)] = e
                    idx1_v[pl.ds(jnp.int32(c * _P) + off, 16)] = e + jnp.int32(128)
                    c += 1
        return jnp.int32(0)
    return pass1


def _make_pass2(xs, ys, zs, rows0_v, rows1_v, outbuf, res, lvl):
    def pass2(g, _):
        off = g * jnp.int32(16)
        _bx, tx = _floor_f32(xs[pl.ds(off, 16)] * res - 0.5)
        _by, ty = _floor_f32(ys[pl.ds(off, 16)] * res - 0.5)
        _bz, tz = _floor_f32(zs[pl.ds(off, 16)] * res - 0.5)
        wx = (1.0 - tx, tx)
        wy = (1.0 - ty, ty)
        wz = (1.0 - tz, tz)
        acc0 = jnp.full((16,), 0.0, jnp.float32)
        acc1 = jnp.full((16,), 0.0, jnp.float32)
        c = 0
        for ox in range(2):
            for oy in range(2):
                wxy = wx[ox] * wy[oy]
                for oz in range(2):
                    w16 = wxy * wz[oz]
                    r0 = rows0_v[pl.ds(jnp.int32(c * _P) + off, 16)]
                    r1 = rows1_v[pl.ds(jnp.int32(c * _P) + off, 16)]
                    acc0 = acc0 + w16 * r0
                    acc1 = acc1 + w16 * r1
                    c += 1
        ot = off & jnp.int32(127)
        opos = jnp.int32(2 * lvl * _P) + ((off - ot) << 1) + ot
        outbuf[pl.ds(opos, 16)] = acc0
        outbuf[pl.ds(opos + jnp.int32(128), 16)] = acc1
        return jnp.int32(0)
    return pass2


def _make_cached_pass(xs, ys, zs, g0, g1, outbuf, res, lvl, gr):
    gr2 = gr * gr
    def cpass(g, _):
        off = g * jnp.int32(16)
        bx, tx = _floor_f32(xs[pl.ds(off, 16)] * res - 0.5)
        by, ty = _floor_f32(ys[pl.ds(off, 16)] * res - 0.5)
        bz, tz = _floor_f32(zs[pl.ds(off, 16)] * res - 0.5)
        lb = ((bx + jnp.int32(1)) * jnp.int32(gr2)
              + (by + jnp.int32(1)) * jnp.int32(gr)
              + (bz + jnp.int32(1)))
        wx = (1.0 - tx, tx)
        wy = (1.0 - ty, ty)
        wz = (1.0 - tz, tz)
        acc0 = jnp.full((16,), 0.0, jnp.float32)
        acc1 = jnp.full((16,), 0.0, jnp.float32)
        for ox in range(2):
            for oy in range(2):
                wxy = wx[ox] * wy[oy]
                idxy = lb + jnp.int32(ox * gr2 + oy * gr)
                for oz in range(2):
                    w16 = wxy * wz[oz]
                    ids = idxy + jnp.int32(oz)
                    acc0 = acc0 + w16 * plsc.load_gather(g0, [ids])
                    acc1 = acc1 + w16 * plsc.load_gather(g1, [ids])
        ot = off & jnp.int32(127)
        opos = jnp.int32(2 * lvl * _P) + ((off - ot) << 1) + ot
        outbuf[pl.ds(opos, 16)] = acc0
        outbuf[pl.ds(opos + jnp.int32(128), 16)] = acc1
        return jnp.int32(0)
    return cpass


def _stage_grids(tab_hbm, idx0_v, idx1_v, grids, sem):
    iota = lax.iota(jnp.int32, 16)
    for lvl in range(_NCACHE):
        gr, sz, m = _GRID_R[lvl], _GRID_SZ[lvl], _GRID_M[lvl]
        g0, g1 = grids[lvl]
        for chunk0 in range(0, sz, _K):
            clen = min(_K, sz - chunk0)
            ngrp = (clen + 15) // 16

            def fill(g, _, chunk0=chunk0, gr=gr, sz=sz, m=m, lvl=lvl):
                n = jnp.int32(chunk0) + g * jnp.int32(16) + iota
                n = jnp.minimum(n, jnp.int32(sz - 1))
                q = (n * jnp.int32(m)) >> jnp.int32(20)
                gz = n - q * jnp.int32(gr)
                q2 = (q * jnp.int32(m)) >> jnp.int32(20)
                gy = q - q2 * jnp.int32(gr)
                gx = q2
                h = _hash3(gx - jnp.int32(1), gy - jnp.int32(1),
                           gz - jnp.int32(1))
                e = _tiled_entry(h, lvl)
                off = g * jnp.int32(16)
                idx0_v[pl.ds(off, 16)] = e
                idx1_v[pl.ds(off, 16)] = e + jnp.int32(128)
                return jnp.int32(0)

            lax.fori_loop(jnp.int32(0), jnp.int32(ngrp), fill, jnp.int32(0),
                          unroll=False)
            cp0 = pltpu.async_copy(tab_hbm.at[idx0_v.at[pl.ds(0, clen)]],
                                   g0.at[pl.ds(chunk0, clen)], sem)
            cp1 = pltpu.async_copy(tab_hbm.at[idx1_v.at[pl.ds(0, clen)]],
                                   g1.at[pl.ds(chunk0, clen)], sem)
            cp0.wait()
            cp1.wait()


def _body(xs_hbm, ys_hbm, zs_hbm, tab_hbm, out_hbm,
          xs, ys, zs,
          idx0_a, idx1_a, rows0_a, rows1_a,
          idx0_b, idx1_b, rows0_b, rows1_b,
          g0_0, g1_0, g0_1, g1_1, g0_2, g1_2,
          outbuf, sem_a, sem_b):
    nc = 2
    wid = lax.axis_index("s") * nc + lax.axis_index("c")
    sets = ((idx0_a, idx1_a, rows0_a, rows1_a, sem_a),
            (idx0_b, idx1_b, rows0_b, rows1_b, sem_b))
    grids = ((g0_0, g1_0), (g0_1, g1_1), (g0_2, g1_2))

    _stage_grids(tab_hbm, idx0_a, idx1_a, grids, sem_a)

    def chunk_body(ch, _):
        base = wid * jnp.int32(_PER_W) + ch * jnp.int32(_P)
        pltpu.sync_copy(xs_hbm.at[pl.ds(base, _P)], xs)
        pltpu.sync_copy(ys_hbm.at[pl.ds(base, _P)], ys)
        pltpu.sync_copy(zs_hbm.at[pl.ds(base, _P)], zs)

        pend = None
        first = True
        for lvl in range(_NCACHE, _NUM_LEVELS):
            idx0_v, idx1_v, rows0_v, rows1_v, sem = sets[lvl % 2]
            pass1 = _make_pass1(xs, ys, zs, idx0_v, idx1_v, _RES[lvl], lvl)
            lax.fori_loop(jnp.int32(0), jnp.int32(_G), pass1, jnp.int32(0),
                          unroll=False)
            cp0 = pltpu.async_copy(tab_hbm.at[idx0_v], rows0_v, sem)
            cp1 = pltpu.async_copy(tab_hbm.at[idx1_v], rows1_v, sem)
            if first:
                for cl in range(_NCACHE):
                    cpass = _make_cached_pass(xs, ys, zs, grids[cl][0],
                                              grids[cl][1], outbuf,
                                              _RES[cl], cl, _GRID_R[cl])
                    lax.fori_loop(jnp.int32(0), jnp.int32(_G), cpass,
                                  jnp.int32(0), unroll=False)
                first = False
            if pend is not None:
                plvl, pcp0, pcp1, prows0, prows1 = pend
                pcp0.wait()
                pcp1.wait()
                pass2 = _make_pass2(xs, ys, zs, prows0, prows1, outbuf,
                                    _RES[plvl], plvl)
                lax.fori_loop(jnp.int32(0), jnp.int32(_G), pass2,
                              jnp.int32(0), unroll=False)
            pend = (lvl, cp0, cp1, rows0_v, rows1_v)

        plvl, pcp0, pcp1, prows0, prows1 = pend
        pcp0.wait()
        pcp1.wait()
        pass2 = _make_pass2(xs, ys, zs, prows0, prows1, outbuf,
                            _RES[plvl], plvl)
        lax.fori_loop(jnp.int32(0), jnp.int32(_G), pass2, jnp.int32(0),
                      unroll=False)

        for lvl in range(_NUM_LEVELS):
            pltpu.sync_copy(
                outbuf.at[pl.ds(lvl * 2 * _P, 2 * _P)],
                out_hbm.at[pl.ds(jnp.int32(lvl * 2 * _N) + base * jnp.int32(2),
                                 2 * _P)])
        return jnp.int32(0)

    lax.fori_loop(jnp.int32(0), jnp.int32(_CHUNKS), chunk_body, jnp.int32(0),
                  unroll=False)


@jax.jit
def kernel(x, tables):
    xt = x.T
    tab = (tables.reshape(_NUM_LEVELS, _HASH_SIZE // 128, 128, _FEAT)
           .transpose(0, 1, 3, 2)
           .reshape(_NUM_LEVELS * _FEAT * _HASH_SIZE))
    buf = lambda dt: pltpu.VMEM((_K,), dt)
    run = pl.kernel(
        _body,
        out_type=jax.ShapeDtypeStruct((_N * 2 * _NUM_LEVELS,), jnp.float32),
        mesh=plsc.VectorSubcoreMesh(core_axis_name="c", subcore_axis_name="s"),
        compiler_params=pltpu.CompilerParams(needs_layout_passes=False),
        scratch_types=[
            pltpu.VMEM((_P,), jnp.float32),
            pltpu.VMEM((_P,), jnp.float32),
            pltpu.VMEM((_P,), jnp.float32),
            buf(jnp.int32), buf(jnp.int32), buf(jnp.float32), buf(jnp.float32),
            buf(jnp.int32), buf(jnp.int32), buf(jnp.float32), buf(jnp.float32),
            pltpu.VMEM((_GRID_PAD[0],), jnp.float32),
            pltpu.VMEM((_GRID_PAD[0],), jnp.float32),
            pltpu.VMEM((_GRID_PAD[1],), jnp.float32),
            pltpu.VMEM((_GRID_PAD[1],), jnp.float32),
            pltpu.VMEM((_GRID_PAD[2],), jnp.float32),
            pltpu.VMEM((_GRID_PAD[2],), jnp.float32),
            pltpu.VMEM((_P * 2 * _NUM_LEVELS,), jnp.float32),
            pltpu.SemaphoreType.DMA,
            pltpu.SemaphoreType.DMA,
        ],
    )
    out = run(xt[0], xt[1], xt[2], tab)
    out4 = out.reshape(_NUM_LEVELS, _N // 128, _FEAT, 128)
    return out4.transpose(1, 3, 0, 2).reshape(_N, _NUM_LEVELS, _FEAT)
